# baseline (speedup 1.0000x reference)
import functools
import jax
import jax.numpy as jnp
from jax import lax
from jax.experimental import pallas as pl
from jax.experimental.pallas import tpu as pltpu
from jax.experimental.pallas import tpu_sc as plsc

HEADS = 4
HIDDEN = 32
FEAT = HEADS * HIDDEN
OUT = 64

NC = 2
NS = 16
CHUNK = 64
HALF = 5632
NL = HALF + 128
PTROWS = 11028
DENROWS = 5648
DENW = DENROWS * HEADS


def _edge_aggregate(sd, ptab, cvec, h1):
    n_chunks_total = sd.shape[0] // (2 * CHUNK)
    chunks_per_tile = n_chunks_total // NS
    npairs = chunks_per_tile // 2
    rows_per_tile = NL // NS

    mesh = plsc.VectorSubcoreMesh(core_axis_name="c", subcore_axis_name="s")

    @functools.partial(
        pl.kernel,
        mesh=mesh,
        compiler_params=pltpu.CompilerParams(needs_layout_passes=False),
        out_type=[
            jax.ShapeDtypeStruct((NC * NL, FEAT), jnp.float32),
            jax.ShapeDtypeStruct((NC * NS * DENW,), jnp.float32),
        ],
        scratch_types=[
            pltpu.VMEM((CHUNK, FEAT), jnp.float32),
            pltpu.VMEM((CHUNK, FEAT), jnp.float32),
            pltpu.VMEM((2 * CHUNK,), jnp.int32),
            pltpu.VMEM((2 * CHUNK,), jnp.int32),
            pltpu.VMEM((CHUNK,), jnp.int32),
            pltpu.VMEM((CHUNK,), jnp.int32),
            pltpu.VMEM((CHUNK * 4,), jnp.float32),
            pltpu.VMEM((PTROWS * 4,), jnp.int32),
            pltpu.VMEM((16,), jnp.float32),
            pltpu.VMEM((DENW,), jnp.float32),
            pltpu.VMEM_SHARED((NL, FEAT), jnp.float32),
            pltpu.SemaphoreType.DMA,
            pltpu.SemaphoreType.DMA,
        ],
    )
    def k(sd_hbm, ptab_hbm, c_hbm, h1_hbm, zrow_hbm, zden_hbm,
          out_hbm, denout_hbm,
          rows0, rows1, sd0, sd1, dstl0, dstl1, exbuf, ptab_v, c_v, den_t,
          out_s, sem0, sem1):
        cid = lax.axis_index("c")
        sid = lax.axis_index("s")

        pltpu.sync_copy(ptab_hbm, ptab_v)
        pltpu.sync_copy(c_hbm, c_v)
        pltpu.sync_copy(zden_hbm, den_t)

        row0 = sid * rows_per_tile
        nz = rows_per_tile // 128
        rem = rows_per_tile - nz * 128
        for z in range(nz):
            pltpu.sync_copy(zrow_hbm, out_s.at[pl.ds(row0 + z * 128, 128)])
        if rem:
            pltpu.sync_copy(zrow_hbm.at[pl.ds(0, rem)],
                            out_s.at[pl.ds(row0 + nz * 128, rem)])

        plsc.subcore_barrier()

        lanes = lax.iota(jnp.int32, 16)
        wrow = lanes // 4
        wcol = lanes - wrow * 4
        cvals = c_v[pl.ds(0, 16)]
        base = sid * chunks_per_tile
        dbase = cid * HALF
        mlo = jnp.int32(-65536)

        slots = ((rows0, sd0, dstl0, sem0), (rows1, sd1, dstl1, sem1))

        def issue(ci, slot):
            rows_v, sd_v, _, sem = slot
            pltpu.sync_copy(sd_hbm.at[pl.ds(ci * (2 * CHUNK), 2 * CHUNK)], sd_v)
            return pltpu.async_copy(h1_hbm.at[sd_v.at[pl.ds(0, CHUNK)]], rows_v, sem)

        def wait_slot(slot):
            rows_v, sd_v, _, sem = slot
            pltpu.make_async_copy(
                h1_hbm.at[sd_v.at[pl.ds(0, CHUNK)]], rows_v, sem).wait()

        def process(slot):
            rows_v, sd_v, dstl_v, _ = slot
            for g in range(CHUNK // 16):
                s16 = sd_v[pl.ds(g * 16, 16)]
                d16 = sd_v[pl.ds(CHUNK + g * 16, 16)]
                dl = d16 - dbase
                okm = (dl >= 0) & (dl < HALF)
                dstl_v[pl.ds(g * 16, 16)] = jnp.where(okm, dl, HALF)
                s4 = s16 * 4
                d4 = d16 * 4
                for h in range(HEADS):
                    ws = plsc.load_gather(ptab_v, [s4 + h])
                    wd = plsc.load_gather(ptab_v, [d4 + h])
                    asv = plsc.bitcast(ws << 16, jnp.float32)
                    adv = plsc.bitcast(wd & mlo, jnp.float32)
                    al = asv + adv
                    al = jnp.where(al >= 0.0, al, al * jnp.float32(0.2))
                    ex = jnp.exp(al - cvals[h])
                    plsc.store_scatter(exbuf, [lanes * 4 + (g * 64 + h)], ex)
            for w in range(CHUNK // 4):
                exw = exbuf[pl.ds(w * 16, 16)]
                dlw = plsc.load_gather(dstl_v, [w * 4 + wrow])
                plsc.addupdate_scatter(den_t, [dlw * 4 + wcol], exw)
                for e4 in range(4):
                    ea = w * 4 + e4
                    for jj in range(FEAT // 16):
                        sc = exw[e4 * 4 + jj // 2]
                        rows_v[ea, pl.ds(jj * 16, 16)] = rows_v[ea, pl.ds(jj * 16, 16)] * sc
            pltpu.sync_copy(rows_v, out_s.at[dstl_v], add=True)

        def body(j, carry):
            issue(base + j, slots[0]).wait()
            process(slots[0])
            return carry

        lax.fori_loop(0, chunks_per_tile, body, 0)

        plsc.subcore_barrier()

        obase = cid * NL + row0
        pltpu.sync_copy(out_s.at[pl.ds(row0, rows_per_tile)],
                        out_hbm.at[pl.ds(obase, rows_per_tile)])
        wid = cid * NS + sid
        pltpu.sync_copy(den_t, denout_hbm.at[pl.ds(wid * DENW, DENW)])

    zrow = jnp.zeros((128, FEAT), jnp.float32)
    zden = jnp.zeros((DENW,), jnp.float32)
    return k(sd, ptab, cvec, h1, zrow, zden)


def kernel(new_node_features, node_features, edge_index, W1, att_src1, att_dst1, b1,
           W2, att_src2, att_dst2, b2, k):
    num_training = node_features.shape[0]
    num_new = new_node_features.shape[0]
    k_static = edge_index.shape[1] // (2 * num_training)
    N = num_training + num_new

    all_nodes = jnp.concatenate([node_features, new_node_features], axis=0)
    tn = node_features / (jnp.linalg.norm(node_features, axis=1, keepdims=True) + 1e-12)
    nn_ = new_node_features / (jnp.linalg.norm(new_node_features, axis=1, keepdims=True) + 1e-12)
    sim = nn_ @ tn.T
    _, topk_idx = jax.lax.top_k(sim, k_static)
    topk_i32 = topk_idx.astype(jnp.int32)

    src0 = edge_index[0].astype(jnp.int32)
    dst0 = edge_index[1].astype(jnp.int32)

    h1 = all_nodes @ W1
    h1r = h1.reshape(N, HEADS, HIDDEN)
    a_src = (h1r * att_src1).sum(-1)
    a_dst = (h1r * att_dst1).sum(-1)
    cvec = jax.nn.leaky_relu(a_src.max(0) + a_dst.max(0), 0.2)
    cpad = jnp.zeros((16,), jnp.float32).at[:HEADS].set(cvec)

    asb = lax.bitcast_convert_type(a_src.astype(jnp.bfloat16), jnp.uint16).astype(jnp.uint32)
    adb = lax.bitcast_convert_type(a_dst.astype(jnp.bfloat16), jnp.uint16).astype(jnp.uint32)
    packed = (asb | (adb << 16)).astype(jnp.int32)
    ptab = jnp.zeros((PTROWS, HEADS), jnp.int32).at[:N].set(packed).reshape(-1)

    new_ids = num_training + jnp.arange(num_new, dtype=jnp.int32)
    tk_flat = topk_i32.reshape(-1)
    rep_new = jnp.repeat(new_ids, k_static)
    sl = jnp.arange(N, dtype=jnp.int32)
    src_all = jnp.concatenate([src0, rep_new, tk_flat, sl])
    dst_all = jnp.concatenate([dst0, tk_flat, rep_new, sl])
    E = src_all.shape[0]
    GRAN = NS * CHUNK * 2
    EPAD = ((E + GRAN - 1) // GRAN) * GRAN
    src_p = jnp.zeros((EPAD,), jnp.int32).at[:E].set(src_all)
    dst_p = jnp.full((EPAD,), N, jnp.int32).at[:E].set(dst_all)
    sd = jnp.stack([src_p.reshape(-1, CHUNK), dst_p.reshape(-1, CHUNK)], axis=1).reshape(-1)

    out_p, den_p = _edge_aggregate(sd, ptab, cpad, h1)
    out_p = out_p.reshape(NC, NL, FEAT)
    den_p = den_p.reshape(NC, NS, DENROWS, HEADS).sum(axis=1)
    num = jnp.concatenate([out_p[0, :HALF], out_p[1, :N - HALF]], axis=0)
    den = jnp.concatenate([den_p[0, :HALF], den_p[1, :N - HALF]], axis=0)
    x1 = num.reshape(N, HEADS, HIDDEN) / (den[..., None] + 1e-16)
    x1 = x1.reshape(N, FEAT) + b1
    x1 = jax.nn.elu(x1)

    kk = k_static + 1
    nbr_f = jnp.concatenate([topk_i32, new_ids[:, None]], axis=1).reshape(-1)
    h2 = x1 @ W2
    a_src2 = (h2 * att_src2[0, 0]).sum(-1)
    a_dst2 = (h2 * att_dst2[0, 0]).sum(-1)
    alpha2 = jax.nn.leaky_relu(a_src2[nbr_f].reshape(num_new, kk) + a_dst2[num_training:, None], 0.2)
    m2 = alpha2.max(axis=1, keepdims=True)
    ex2 = jnp.exp(alpha2 - m2)
    coef2 = ex2 / (ex2.sum(axis=1, keepdims=True) + 1e-16)
    out2 = (h2[nbr_f].reshape(num_new, kk, OUT) * coef2[..., None]).sum(axis=1)
    return out2 + b2


# speedup vs baseline: 1.1042x; 1.1042x over previous
import functools
import jax
import jax.numpy as jnp
from jax import lax
from jax.experimental import pallas as pl
from jax.experimental.pallas import tpu as pltpu
from jax.experimental.pallas import tpu_sc as plsc

HEADS = 4
HIDDEN = 32
FEAT = HEADS * HIDDEN
OUT = 64

NC = 2
NS = 16
CHUNK = 128
HALF = 5632
NL = HALF + 128
PTROWS = 11028
DENROWS = 5648
DENW = DENROWS * HEADS


def _edge_aggregate(sd, ptab, cvec, h1):
    n_chunks_total = sd.shape[0] // (2 * CHUNK)
    chunks_per_tile = n_chunks_total // NS
    npairs = chunks_per_tile // 2
    rows_per_tile = NL // NS

    mesh = plsc.VectorSubcoreMesh(core_axis_name="c", subcore_axis_name="s")

    @functools.partial(
        pl.kernel,
        mesh=mesh,
        compiler_params=pltpu.CompilerParams(needs_layout_passes=False),
        out_type=[
            jax.ShapeDtypeStruct((NC * NL, FEAT), jnp.float32),
            jax.ShapeDtypeStruct((NC * NS * DENW,), jnp.float32),
        ],
        scratch_types=[
            pltpu.VMEM((CHUNK, FEAT), jnp.float32),
            pltpu.VMEM((2 * CHUNK,), jnp.int32),
            pltpu.VMEM((CHUNK,), jnp.int32),
            pltpu.VMEM((CHUNK * 4,), jnp.float32),
            pltpu.VMEM((PTROWS * 4,), jnp.int32),
            pltpu.VMEM((16,), jnp.float32),
            pltpu.VMEM((DENW,), jnp.float32),
            pltpu.VMEM_SHARED((NL, FEAT), jnp.float32),
            pltpu.SemaphoreType.DMA,
            pltpu.SemaphoreType.DMA,
        ],
    )
    def k(sd_hbm, ptab_hbm, c_hbm, h1_hbm, zrow_hbm, zden_hbm,
          out_hbm, denout_hbm,
          rows0, sd0, dstl0, exbuf, ptab_v, c_v, den_t,
          out_s, sem0, sem1):
        cid = lax.axis_index("c")
        sid = lax.axis_index("s")

        pltpu.sync_copy(ptab_hbm, ptab_v)
        pltpu.sync_copy(c_hbm, c_v)
        pltpu.sync_copy(zden_hbm, den_t)

        row0 = sid * rows_per_tile
        nz = rows_per_tile // 128
        rem = rows_per_tile - nz * 128
        for z in range(nz):
            pltpu.sync_copy(zrow_hbm, out_s.at[pl.ds(row0 + z * 128, 128)])
        if rem:
            pltpu.sync_copy(zrow_hbm.at[pl.ds(0, rem)],
                            out_s.at[pl.ds(row0 + nz * 128, rem)])

        plsc.subcore_barrier()

        lanes = lax.iota(jnp.int32, 16)
        wrow = lanes // 4
        wcol = lanes - wrow * 4
        cvals = c_v[pl.ds(0, 16)]
        base = sid * chunks_per_tile
        dbase = cid * HALF
        mlo = jnp.int32(-65536)

        slots = ((rows0, sd0, dstl0, sem0),)

        def issue(ci, slot):
            rows_v, sd_v, _, sem = slot
            pltpu.sync_copy(sd_hbm.at[pl.ds(ci * (2 * CHUNK), 2 * CHUNK)], sd_v)
            return pltpu.async_copy(h1_hbm.at[sd_v.at[pl.ds(0, CHUNK)]], rows_v, sem)

        def wait_slot(slot):
            rows_v, sd_v, _, sem = slot
            pltpu.make_async_copy(
                h1_hbm.at[sd_v.at[pl.ds(0, CHUNK)]], rows_v, sem).wait()

        def process(slot):
            rows_v, sd_v, dstl_v, _ = slot
            for g in range(CHUNK // 16):
                s16 = sd_v[pl.ds(g * 16, 16)]
                d16 = sd_v[pl.ds(CHUNK + g * 16, 16)]
                dl = d16 - dbase
                okm = (dl >= 0) & (dl < HALF)
                dstl_v[pl.ds(g * 16, 16)] = jnp.where(okm, dl, HALF)
                s4 = s16 * 4
                d4 = d16 * 4
                for h in range(HEADS):
                    ws = plsc.load_gather(ptab_v, [s4 + h])
                    wd = plsc.load_gather(ptab_v, [d4 + h])
                    asv = plsc.bitcast(ws << 16, jnp.float32)
                    adv = plsc.bitcast(wd & mlo, jnp.float32)
                    al = asv + adv
                    al = jnp.where(al >= 0.0, al, al * jnp.float32(0.2))
                    ex = jnp.exp(al - cvals[h])
                    plsc.store_scatter(exbuf, [lanes * 4 + (g * 64 + h)], ex)
            for w in range(CHUNK // 4):
                exw = exbuf[pl.ds(w * 16, 16)]
                dlw = plsc.load_gather(dstl_v, [w * 4 + wrow])
                plsc.addupdate_scatter(den_t, [dlw * 4 + wcol], exw)
                for e4 in range(4):
                    ea = w * 4 + e4
                    for jj in range(FEAT // 16):
                        sc = exw[e4 * 4 + jj // 2]
                        rows_v[ea, pl.ds(jj * 16, 16)] = rows_v[ea, pl.ds(jj * 16, 16)] * sc
            pltpu.sync_copy(rows_v, out_s.at[dstl_v], add=True)

        def body(j, carry):
            issue(base + j, slots[0]).wait()
            process(slots[0])
            return carry

        lax.fori_loop(0, chunks_per_tile, body, 0)

        plsc.subcore_barrier()

        obase = cid * NL + row0
        pltpu.sync_copy(out_s.at[pl.ds(row0, rows_per_tile)],
                        out_hbm.at[pl.ds(obase, rows_per_tile)])
        wid = cid * NS + sid
        pltpu.sync_copy(den_t, denout_hbm.at[pl.ds(wid * DENW, DENW)])

    zrow = jnp.zeros((128, FEAT), jnp.float32)
    zden = jnp.zeros((DENW,), jnp.float32)
    return k(sd, ptab, cvec, h1, zrow, zden)


def kernel(new_node_features, node_features, edge_index, W1, att_src1, att_dst1, b1,
           W2, att_src2, att_dst2, b2, k):
    num_training = node_features.shape[0]
    num_new = new_node_features.shape[0]
    k_static = edge_index.shape[1] // (2 * num_training)
    N = num_training + num_new

    all_nodes = jnp.concatenate([node_features, new_node_features], axis=0)
    tn = node_features / (jnp.linalg.norm(node_features, axis=1, keepdims=True) + 1e-12)
    nn_ = new_node_features / (jnp.linalg.norm(new_node_features, axis=1, keepdims=True) + 1e-12)
    sim = nn_ @ tn.T
    _, topk_idx = jax.lax.top_k(sim, k_static)
    topk_i32 = topk_idx.astype(jnp.int32)

    src0 = edge_index[0].astype(jnp.int32)
    dst0 = edge_index[1].astype(jnp.int32)

    h1 = all_nodes @ W1
    h1r = h1.reshape(N, HEADS, HIDDEN)
    a_src = (h1r * att_src1).sum(-1)
    a_dst = (h1r * att_dst1).sum(-1)
    cvec = jax.nn.leaky_relu(a_src.max(0) + a_dst.max(0), 0.2)
    cpad = jnp.zeros((16,), jnp.float32).at[:HEADS].set(cvec)

    asb = lax.bitcast_convert_type(a_src.astype(jnp.bfloat16), jnp.uint16).astype(jnp.uint32)
    adb = lax.bitcast_convert_type(a_dst.astype(jnp.bfloat16), jnp.uint16).astype(jnp.uint32)
    packed = (asb | (adb << 16)).astype(jnp.int32)
    ptab = jnp.zeros((PTROWS, HEADS), jnp.int32).at[:N].set(packed).reshape(-1)

    new_ids = num_training + jnp.arange(num_new, dtype=jnp.int32)
    tk_flat = topk_i32.reshape(-1)
    rep_new = jnp.repeat(new_ids, k_static)
    sl = jnp.arange(N, dtype=jnp.int32)
    src_all = jnp.concatenate([src0, rep_new, tk_flat, sl])
    dst_all = jnp.concatenate([dst0, tk_flat, rep_new, sl])
    E = src_all.shape[0]
    GRAN = NS * CHUNK * 2
    EPAD = ((E + GRAN - 1) // GRAN) * GRAN
    src_p = jnp.zeros((EPAD,), jnp.int32).at[:E].set(src_all)
    dst_p = jnp.full((EPAD,), N, jnp.int32).at[:E].set(dst_all)
    sd = jnp.stack([src_p.reshape(-1, CHUNK), dst_p.reshape(-1, CHUNK)], axis=1).reshape(-1)

    out_p, den_p = _edge_aggregate(sd, ptab, cpad, h1)
    out_p = out_p.reshape(NC, NL, FEAT)
    den_p = den_p.reshape(NC, NS, DENROWS, HEADS).sum(axis=1)
    num = jnp.concatenate([out_p[0, :HALF], out_p[1, :N - HALF]], axis=0)
    den = jnp.concatenate([den_p[0, :HALF], den_p[1, :N - HALF]], axis=0)
    x1 = num.reshape(N, HEADS, HIDDEN) / (den[..., None] + 1e-16)
    x1 = x1.reshape(N, FEAT) + b1
    x1 = jax.nn.elu(x1)

    kk = k_static + 1
    nbr_f = jnp.concatenate([topk_i32, new_ids[:, None]], axis=1).reshape(-1)
    h2 = x1 @ W2
    a_src2 = (h2 * att_src2[0, 0]).sum(-1)
    a_dst2 = (h2 * att_dst2[0, 0]).sum(-1)
    alpha2 = jax.nn.leaky_relu(a_src2[nbr_f].reshape(num_new, kk) + a_dst2[num_training:, None], 0.2)
    m2 = alpha2.max(axis=1, keepdims=True)
    ex2 = jnp.exp(alpha2 - m2)
    coef2 = ex2 / (ex2.sum(axis=1, keepdims=True) + 1e-16)
    out2 = (h2[nbr_f].reshape(num_new, kk, OUT) * coef2[..., None]).sum(axis=1)
    return out2 + b2
